# baseline (speedup 1.0000x reference)
import jax
import jax.numpy as jnp
from jax.experimental import pallas as pl
from jax.experimental.pallas import tpu as pltpu

_N_CHUNKS = 4


def _dma_kernel(emb_ref, out_ref, sems):
    rows = emb_ref.shape[0]
    chunk = rows // _N_CHUNKS
    copies = [
        pltpu.make_async_copy(
            emb_ref.at[pl.ds(i * chunk, chunk), :],
            out_ref.at[0, pl.ds(i * chunk, chunk), :],
            sems.at[i],
        )
        for i in range(_N_CHUNKS)
    ]
    for c in copies:
        c.start()
    for c in copies:
        c.wait()


def kernel(x, emb):
    T = x.shape[1]
    D = emb.shape[1]
    out = pl.pallas_call(
        _dma_kernel,
        in_specs=[pl.BlockSpec(memory_space=pl.ANY)],
        out_specs=pl.BlockSpec(memory_space=pl.ANY),
        out_shape=jax.ShapeDtypeStruct((1, T, D), emb.dtype),
        scratch_shapes=[pltpu.SemaphoreType.DMA((_N_CHUNKS,))],
    )(emb[:T])
    return out


# speedup vs baseline: 42.7307x; 42.7307x over previous
import jax
import jax.numpy as jnp
from jax.experimental import pallas as pl
from jax.experimental.pallas import tpu as pltpu

_BLOCK = 1024


def _copy_block(emb_ref, out_ref):
    out_ref[0, :, :] = emb_ref[:, :]


def kernel(x, emb):
    T = x.shape[1]
    D = emb.shape[1]
    assert T % _BLOCK == 0
    out = pl.pallas_call(
        _copy_block,
        grid=(T // _BLOCK,),
        in_specs=[pl.BlockSpec((_BLOCK, D), lambda i: (i, 0))],
        out_specs=pl.BlockSpec((1, _BLOCK, D), lambda i: (0, i, 0)),
        out_shape=jax.ShapeDtypeStruct((1, T, D), emb.dtype),
        compiler_params=pltpu.CompilerParams(
            dimension_semantics=("parallel",),
        ),
    )(emb[:T])
    return out


# speedup vs baseline: 46.1338x; 1.0796x over previous
import jax
import jax.numpy as jnp
from jax.experimental import pallas as pl
from jax.experimental.pallas import tpu as pltpu

_BLOCK = 2048


def _copy_block(emb_ref, out_ref):
    out_ref[0, :, :] = emb_ref[:, :]


def kernel(x, emb):
    T = x.shape[1]
    D = emb.shape[1]
    assert T % _BLOCK == 0
    out = pl.pallas_call(
        _copy_block,
        grid=(T // _BLOCK,),
        in_specs=[pl.BlockSpec((_BLOCK, D), lambda i: (i, 0))],
        out_specs=pl.BlockSpec((1, _BLOCK, D), lambda i: (0, i, 0)),
        out_shape=jax.ShapeDtypeStruct((1, T, D), emb.dtype),
        compiler_params=pltpu.CompilerParams(
            dimension_semantics=("parallel",),
        ),
    )(emb[:T])
    return out


# speedup vs baseline: 48.7417x; 1.0565x over previous
import jax
import jax.numpy as jnp
from jax.experimental import pallas as pl
from jax.experimental.pallas import tpu as pltpu

_BLOCK = 4096


def _copy_block(emb_ref, out_ref):
    out_ref[0, :, :] = emb_ref[:, :]


def kernel(x, emb):
    T = x.shape[1]
    D = emb.shape[1]
    assert T % _BLOCK == 0
    out = pl.pallas_call(
        _copy_block,
        grid=(T // _BLOCK,),
        in_specs=[pl.BlockSpec((_BLOCK, D), lambda i: (i, 0))],
        out_specs=pl.BlockSpec((1, _BLOCK, D), lambda i: (0, i, 0)),
        out_shape=jax.ShapeDtypeStruct((1, T, D), emb.dtype),
        compiler_params=pltpu.CompilerParams(
            dimension_semantics=("parallel",),
        ),
    )(emb[:T])
    return out
